# baseline (speedup 1.0000x reference)
import functools

import jax
import jax.numpy as jnp
from jax import lax
from jax.experimental import pallas as pl
from jax.experimental.pallas import tpu as pltpu
from jax.experimental.pallas import tpu_sc as plsc

V = 1000000
VH = V // 2
D = 64
B_POS = 16384
B_NEG = 81920
TOTAL = B_POS + B_NEG

NC, NS = 2, 16
NW = NC * NS
POS_PW = B_POS // NW
NEG_PW = B_NEG // NW
PAIRS_PW = POS_PW + NEG_PW
C = 64
NCH = PAIRS_PW // C
G = C // 16


BT = 4096
_GRID_T = (V + BT - 1) // BT


NFULL = V // 128
CPW = 245

_pack_mesh = plsc.VectorSubcoreMesh(
    core_axis_name="c", subcore_axis_name="s", num_cores=NC, num_subcores=NS)


@functools.partial(
    pl.kernel,
    out_type=(jax.ShapeDtypeStruct((VH, 128), jnp.float32),
              jax.ShapeDtypeStruct((VH, 128), jnp.float32)),
    mesh=_pack_mesh,
    compiler_params=pltpu.CompilerParams(needs_layout_passes=False),
    scratch_types=[
        pltpu.VMEM((D, 128), jnp.float32),
        pltpu.VMEM((D, 128), jnp.float32),
        pltpu.VMEM((D, 128), jnp.float32),
        pltpu.VMEM((D, 128), jnp.float32),
        pltpu.SemaphoreType.DMA,
        pltpu.SemaphoreType.DMA,
        pltpu.SemaphoreType.DMA,
        pltpu.SemaphoreType.DMA,
    ],
)
def _pack_sc(ctx_t, tgt_t, octx, otgt,
             tb_a, tb_b, pb_a, pb_b, si_a, si_b, so_a, so_b):
    wid = lax.axis_index("s") * NC + lax.axis_index("c")
    iota = lax.iota(jnp.int32, 16)
    rows_q = [iota + 16 * q for q in range(4)]
    cbeg = wid * CPW
    cend = jnp.minimum(cbeg + CPW, NFULL)

    for src, out in ((ctx_t, octx), (tgt_t, otgt)):

        def issue_in(c, tb, sem):
            base = pl.multiple_of(c * 128, 128)
            for a in range(8):
                pltpu.async_copy(src.at[pl.ds(a * 8, 8), pl.ds(base, 128)],
                                 tb.at[pl.ds(a * 8, 8), :], sem)

        def drain_in(tb, sem):
            pltpu.make_async_copy(
                src.at[pl.ds(0, D), pl.ds(0, 128)], tb, sem).wait()

        def drain_out(pb, sem):
            pltpu.make_async_copy(
                src.at[pl.ds(0, D), pl.ds(0, 128)], pb, sem).wait()

        def transpose_out(c, tb, pb, sem):
            def tbody(k2, carry):
                ce = jnp.zeros((16,), jnp.int32) + 2 * k2
                co = ce + 1
                for q in range(4):
                    pb[k2, pl.ds(q * 16, 16)] = plsc.load_gather(
                        tb, [rows_q[q], ce])
                    pb[k2, pl.ds(64 + q * 16, 16)] = plsc.load_gather(
                        tb, [rows_q[q], co])
                return carry

            lax.fori_loop(0, D, tbody, 0)
            obase = pl.multiple_of(c * 64, 64)
            pltpu.async_copy(pb, out.at[pl.ds(obase, 64), :], sem)

        issue_in(cbeg, tb_a, si_a)

        def body(i, carry):
            c0 = cbeg + 2 * i

            @pl.when(c0 + 1 < cend)
            def _():
                issue_in(c0 + 1, tb_b, si_b)

            @pl.when(c0 < cend)
            def _():
                drain_in(tb_a, si_a)

                @pl.when(i > 0)
                def _():
                    drain_out(pb_a, so_a)

                transpose_out(c0, tb_a, pb_a, so_a)

            @pl.when(c0 + 2 < cend)
            def _():
                issue_in(c0 + 2, tb_a, si_a)

            @pl.when(c0 + 1 < cend)
            def _():
                drain_in(tb_b, si_b)

                @pl.when(i > 0)
                def _():
                    drain_out(pb_b, so_b)

                transpose_out(c0 + 1, tb_b, pb_b, so_b)

            return carry

        lax.fori_loop(0, (CPW + 1) // 2, body, 0)
        drain_out(pb_a, so_a)
        drain_out(pb_b, so_b)


def _tail_tc(xc_ref, pc_ref, xt_ref, pt_ref, oc_ref, ot_ref):
    for x_ref, o_ref in ((xc_ref, oc_ref), (xt_ref, ot_ref)):
        x = x_ref[...][:, 0:64]
        xt = x.T
        x3 = xt.reshape(32, 2, D)
        o_ref[...] = jnp.concatenate([x3[:, 0, :], x3[:, 1, :]], axis=1)


_tail_call = pl.pallas_call(
    _tail_tc,
    grid=(1,),
    in_specs=[
        pl.BlockSpec((D, 128), lambda g: (0, NFULL)),
        pl.BlockSpec(memory_space=pl.ANY),
        pl.BlockSpec((D, 128), lambda g: (0, NFULL)),
        pl.BlockSpec(memory_space=pl.ANY),
    ],
    out_specs=[
        pl.BlockSpec((32, 128), lambda g: (NFULL * 2, 0)),
        pl.BlockSpec((32, 128), lambda g: (NFULL * 2, 0)),
    ],
    out_shape=[jax.ShapeDtypeStruct((VH, 128), jnp.float32),
               jax.ShapeDtypeStruct((VH, 128), jnp.float32)],
    input_output_aliases={1: 0, 3: 1},
)


_mesh = plsc.VectorSubcoreMesh(
    core_axis_name="c", subcore_axis_name="s", num_cores=NC, num_subcores=NS)


@functools.partial(
    pl.kernel,
    out_type=jax.ShapeDtypeStruct((TOTAL,), jnp.float32),
    mesh=_mesh,
    compiler_params=pltpu.CompilerParams(needs_layout_passes=False),
    scratch_types=[
        pltpu.VMEM((PAIRS_PW,), jnp.int32),
        pltpu.VMEM((PAIRS_PW,), jnp.int32),
        pltpu.VMEM((PAIRS_PW,), jnp.int32),
        pltpu.VMEM((PAIRS_PW,), jnp.int32),
        pltpu.VMEM((C, 128), jnp.float32),
        pltpu.VMEM((C, 128), jnp.float32),
        pltpu.VMEM((C, 128), jnp.float32),
        pltpu.VMEM((C, 128), jnp.float32),
        pltpu.VMEM((PAIRS_PW,), jnp.float32),
        pltpu.SemaphoreType.DMA,
        pltpu.SemaphoreType.DMA,
    ],
)
def _dots_sc(pc, pt, ncx, ntg, ctxp, tgtp, out,
             idx_c, idx_t, idh_c, idh_t, rc_a, rt_a, rc_b, rt_b,
             dots_v, sem_a, sem_b):
    wid = lax.axis_index("s") * NC + lax.axis_index("c")
    lanes = lax.iota(jnp.int32, 16)

    pltpu.sync_copy(pc.at[pl.ds(wid * POS_PW, POS_PW)],
                    idx_c.at[pl.ds(0, POS_PW)])
    pltpu.sync_copy(ncx.at[pl.ds(wid * NEG_PW, NEG_PW)],
                    idx_c.at[pl.ds(POS_PW, NEG_PW)])
    pltpu.sync_copy(pt.at[pl.ds(wid * POS_PW, POS_PW)],
                    idx_t.at[pl.ds(0, POS_PW)])
    pltpu.sync_copy(ntg.at[pl.ds(wid * NEG_PW, NEG_PW)],
                    idx_t.at[pl.ds(POS_PW, NEG_PW)])

    def halve_body(g, carry):
        idh_c[pl.ds(g * 16, 16)] = lax.shift_right_logical(
            idx_c[pl.ds(g * 16, 16)], 1)
        idh_t[pl.ds(g * 16, 16)] = lax.shift_right_logical(
            idx_t[pl.ds(g * 16, 16)], 1)
        return carry

    lax.fori_loop(0, PAIRS_PW // 16, halve_body, 0)

    def issue(k, rc, rt, sem):
        pltpu.async_copy(ctxp.at[idh_c.at[pl.ds(k * C, C)]], rc, sem)
        pltpu.async_copy(tgtp.at[idh_t.at[pl.ds(k * C, C)]], rt, sem)

    def drain(rc, rt, sem):
        pltpu.make_async_copy(ctxp.at[pl.ds(0, C), :], rc, sem).wait()
        pltpu.make_async_copy(ctxp.at[pl.ds(0, C), :], rt, sem).wait()

    def compute(k, rc, rt):
        for g in range(G):
            vc = idx_c[pl.ds(k * C + g * 16, 16)]
            vt = idx_t[pl.ds(k * C + g * 16, 16)]
            tot = jnp.zeros((16,), jnp.float32)
            for i in range(16):
                j = g * 16 + i
                hc = (vc[i] & 1) * D
                ht = (vt[i] & 1) * D
                acc = rc[j, pl.ds(hc, 16)] * rt[j, pl.ds(ht, 16)]
                for q in range(1, 4):
                    acc = acc + (rc[j, pl.ds(hc + q * 16, 16)]
                                 * rt[j, pl.ds(ht + q * 16, 16)])
                s = jnp.sum(acc)
                tot = jnp.where(lanes == i, s, tot)
            dots_v[pl.ds(k * C + g * 16, 16)] = tot

    issue(0, rc_a, rt_a, sem_a)

    def body(k2, carry):
        c0 = k2 * 2
        issue(c0 + 1, rc_b, rt_b, sem_b)
        drain(rc_a, rt_a, sem_a)
        compute(c0, rc_a, rt_a)

        @pl.when(k2 < NCH // 2 - 1)
        def _():
            issue(c0 + 2, rc_a, rt_a, sem_a)

        drain(rc_b, rt_b, sem_b)
        compute(c0 + 1, rc_b, rt_b)
        return carry

    lax.fori_loop(0, NCH // 2, body, 0)

    pltpu.sync_copy(dots_v.at[pl.ds(0, POS_PW)],
                    out.at[pl.ds(wid * POS_PW, POS_PW)])
    pltpu.sync_copy(dots_v.at[pl.ds(POS_PW, NEG_PW)],
                    out.at[pl.ds(B_POS + wid * NEG_PW, NEG_PW)])


def _loss_tc(dp_ref, dn_ref, out_ref):
    dp = dp_ref[...]
    dn = dn_ref[...]
    pos_loss = -jnp.mean(jnp.log(jax.nn.sigmoid(dp)))
    neg_loss = -jnp.mean(jnp.log(jax.nn.sigmoid(-dn)))
    out_ref[0, 0] = pos_loss + neg_loss


_loss_call = pl.pallas_call(
    _loss_tc,
    out_shape=jax.ShapeDtypeStruct((1, 1), jnp.float32),
    out_specs=pl.BlockSpec(memory_space=pltpu.SMEM),
)


def kernel(positive_context, positive_target, negative_context,
           negative_target, context_embeddings, target_embeddings):
    pc = positive_context.astype(jnp.int32)
    pt = positive_target.astype(jnp.int32)
    ncx = negative_context.astype(jnp.int32)
    ntg = negative_target.astype(jnp.int32)
    ctx_t = context_embeddings.T
    tgt_t = target_embeddings.T
    ctxp0, tgtp0 = _pack_sc(ctx_t, tgt_t)
    ctxp, tgtp = _tail_call(ctx_t, ctxp0, tgt_t, tgtp0)
    dots = _dots_sc(pc, pt, ncx, ntg, ctxp, tgtp)
    dp = dots[:B_POS].reshape(B_POS // 128, 128)
    dn = dots[B_POS:].reshape(B_NEG // 128, 128)
    return _loss_call(dp, dn)[0, 0]


# speedup vs baseline: 3.3707x; 3.3707x over previous
import functools

import jax
import jax.numpy as jnp
from jax import lax
from jax.experimental import pallas as pl
from jax.experimental.pallas import tpu as pltpu
from jax.experimental.pallas import tpu_sc as plsc

V = 1000000
VH = V // 2
D = 64
B_POS = 16384
B_NEG = 81920
TOTAL = B_POS + B_NEG

NC, NS = 2, 16
NW = NC * NS
POS_PW = B_POS // NW
NEG_PW = B_NEG // NW
PAIRS_PW = POS_PW + NEG_PW
C = 64
NCH = PAIRS_PW // C
G = C // 16


BT = 4096
_GRID_T = (V + BT - 1) // BT


def _pack_tc(x_ref, o_ref):
    x = x_ref[...]
    xt = x.T
    x3 = xt.reshape(BT // 2, 2, D)
    o_ref[...] = jnp.concatenate([x3[:, 0, :], x3[:, 1, :]], axis=1)


_pack_call = pl.pallas_call(
    _pack_tc,
    grid=(_GRID_T,),
    in_specs=[pl.BlockSpec((D, BT), lambda g: (0, g))],
    out_specs=pl.BlockSpec((BT // 2, 128), lambda g: (g, 0)),
    out_shape=jax.ShapeDtypeStruct((VH, 128), jnp.float32),
)


_mesh = plsc.VectorSubcoreMesh(
    core_axis_name="c", subcore_axis_name="s", num_cores=NC, num_subcores=NS)


@functools.partial(
    pl.kernel,
    out_type=jax.ShapeDtypeStruct((TOTAL,), jnp.float32),
    mesh=_mesh,
    compiler_params=pltpu.CompilerParams(needs_layout_passes=False),
    scratch_types=[
        pltpu.VMEM((PAIRS_PW,), jnp.int32),
        pltpu.VMEM((PAIRS_PW,), jnp.int32),
        pltpu.VMEM((PAIRS_PW,), jnp.int32),
        pltpu.VMEM((PAIRS_PW,), jnp.int32),
        pltpu.VMEM((C, 128), jnp.float32),
        pltpu.VMEM((C, 128), jnp.float32),
        pltpu.VMEM((C, 128), jnp.float32),
        pltpu.VMEM((C, 128), jnp.float32),
        pltpu.VMEM((PAIRS_PW,), jnp.float32),
        pltpu.SemaphoreType.DMA,
        pltpu.SemaphoreType.DMA,
    ],
)
def _dots_sc(pc, pt, ncx, ntg, ctxp, tgtp, out,
             idx_c, idx_t, idh_c, idh_t, rc_a, rt_a, rc_b, rt_b,
             dots_v, sem_a, sem_b):
    wid = lax.axis_index("s") * NC + lax.axis_index("c")
    lanes = lax.iota(jnp.int32, 16)

    pltpu.sync_copy(pc.at[pl.ds(wid * POS_PW, POS_PW)],
                    idx_c.at[pl.ds(0, POS_PW)])
    pltpu.sync_copy(ncx.at[pl.ds(wid * NEG_PW, NEG_PW)],
                    idx_c.at[pl.ds(POS_PW, NEG_PW)])
    pltpu.sync_copy(pt.at[pl.ds(wid * POS_PW, POS_PW)],
                    idx_t.at[pl.ds(0, POS_PW)])
    pltpu.sync_copy(ntg.at[pl.ds(wid * NEG_PW, NEG_PW)],
                    idx_t.at[pl.ds(POS_PW, NEG_PW)])

    def halve_body(g, carry):
        idh_c[pl.ds(g * 16, 16)] = lax.shift_right_logical(
            idx_c[pl.ds(g * 16, 16)], 1)
        idh_t[pl.ds(g * 16, 16)] = lax.shift_right_logical(
            idx_t[pl.ds(g * 16, 16)], 1)
        return carry

    lax.fori_loop(0, PAIRS_PW // 16, halve_body, 0)

    def issue(k, rc, rt, sem):
        pltpu.async_copy(ctxp.at[idh_c.at[pl.ds(k * C, C)]], rc, sem)
        pltpu.async_copy(tgtp.at[idh_t.at[pl.ds(k * C, C)]], rt, sem)

    def drain(rc, rt, sem):
        pltpu.make_async_copy(ctxp.at[pl.ds(0, C), :], rc, sem).wait()
        pltpu.make_async_copy(ctxp.at[pl.ds(0, C), :], rt, sem).wait()

    def compute(k, rc, rt):
        for g in range(G):
            vc = idx_c[pl.ds(k * C + g * 16, 16)]
            vt = idx_t[pl.ds(k * C + g * 16, 16)]
            tot = jnp.zeros((16,), jnp.float32)
            for i in range(16):
                j = g * 16 + i
                hc = (vc[i] & 1) * D
                ht = (vt[i] & 1) * D
                acc = rc[j, pl.ds(hc, 16)] * rt[j, pl.ds(ht, 16)]
                for q in range(1, 4):
                    acc = acc + (rc[j, pl.ds(hc + q * 16, 16)]
                                 * rt[j, pl.ds(ht + q * 16, 16)])
                s = jnp.sum(acc)
                tot = jnp.where(lanes == i, s, tot)
            dots_v[pl.ds(k * C + g * 16, 16)] = tot

    issue(0, rc_a, rt_a, sem_a)

    def body(k2, carry):
        c0 = k2 * 2
        issue(c0 + 1, rc_b, rt_b, sem_b)
        drain(rc_a, rt_a, sem_a)
        compute(c0, rc_a, rt_a)

        @pl.when(k2 < NCH // 2 - 1)
        def _():
            issue(c0 + 2, rc_a, rt_a, sem_a)

        drain(rc_b, rt_b, sem_b)
        compute(c0 + 1, rc_b, rt_b)
        return carry

    lax.fori_loop(0, NCH // 2, body, 0)

    pltpu.sync_copy(dots_v.at[pl.ds(0, POS_PW)],
                    out.at[pl.ds(wid * POS_PW, POS_PW)])
    pltpu.sync_copy(dots_v.at[pl.ds(POS_PW, NEG_PW)],
                    out.at[pl.ds(B_POS + wid * NEG_PW, NEG_PW)])


def _loss_tc(dp_ref, dn_ref, out_ref):
    dp = dp_ref[...]
    dn = dn_ref[...]
    pos_loss = -jnp.mean(jnp.log(jax.nn.sigmoid(dp)))
    neg_loss = -jnp.mean(jnp.log(jax.nn.sigmoid(-dn)))
    out_ref[0, 0] = pos_loss + neg_loss


_loss_call = pl.pallas_call(
    _loss_tc,
    out_shape=jax.ShapeDtypeStruct((1, 1), jnp.float32),
    out_specs=pl.BlockSpec(memory_space=pltpu.SMEM),
)


def kernel(positive_context, positive_target, negative_context,
           negative_target, context_embeddings, target_embeddings):
    pc = positive_context.astype(jnp.int32)
    pt = positive_target.astype(jnp.int32)
    ncx = negative_context.astype(jnp.int32)
    ntg = negative_target.astype(jnp.int32)
    ctxp = _pack_call(context_embeddings.T)
    tgtp = _pack_call(target_embeddings.T)
    dots = _dots_sc(pc, pt, ncx, ntg, ctxp, tgtp)
    dp = dots[:B_POS].reshape(B_POS // 128, 128)
    dn = dots[B_POS:].reshape(B_NEG // 128, 128)
    return _loss_call(dp, dn)[0, 0]
